# baseline (speedup 1.0000x reference)
import functools

import jax
import jax.numpy as jnp
from jax import lax
from jax.experimental import pallas as pl
from jax.experimental.pallas import tpu as pltpu
from jax.experimental.pallas import tpu_sc as plsc

EPS = 1e-5
LOG_EPS = -11.512925464970229
NPTS = 100
TOP_N = 8
NEG = -1.0e30
IBIG = 2**30

NC, NS, L = 2, 16, 16
NW = NC * NS
B, D, M, R, M_OUT = 1024, 8, 5, 16384, 5
ROWS_PER_W = B // NW
MM = M + 1
NGRP = R // L
KSUP = 8
NSUP = NGRP // KSUP

_GATHER_DNUMS = lax.GatherDimensionNumbers(
    offset_dims=(), collapsed_slice_dims=(0,), start_index_map=(0,))


def _vtake(v, idx):
    return lax.gather(v, idx[:, None], _GATHER_DNUMS, (1,),
                      mode=lax.GatherScatterMode.PROMISE_IN_BOUNDS)


def _butterfly(v, op):
    lane = lax.broadcasted_iota(jnp.int32, (L,), 0)
    for sh in (8, 4, 2, 1):
        v = op(v, _vtake(v, lane ^ sh))
    return v


def _tc_body(x_ref, cs_ref, ss_ref, ant_ref, out_ref, cm_ref, oh_ref,
             *, bt, r):
    @pl.when(pl.program_id(0) == 0)
    def _build_onehot():
        for i in range(D):
            ai = ant_ref[i:i + 1, :]
            ai = jnp.where(ai < 0, M, ai)
            js = jax.lax.broadcasted_iota(jnp.int32, (MM, r), 0)
            oh_ref[i * MM:(i + 1) * MM, :] = (js == ai).astype(jnp.float32)

    x = x_ref[...]
    pieces = []
    for i in range(D):
        xi = x[:, i:i + 1]
        ci = cs_ref[i:i + 1, :]
        si = ss_ref[i:i + 1, :]
        t = -((xi - ci) ** 2) / (2.0 * si * si)
        t = jnp.clip(t, LOG_EPS, 0.0)
        pieces.append(t)
        pieces.append(jnp.zeros((bt, 1), jnp.float32))
    logF = jnp.concatenate(pieces, axis=1)

    lg = jax.lax.dot_general(
        logF, oh_ref[...], (((1,), (0,)), ((), ())),
        precision=jax.lax.Precision.HIGHEST,
        preferred_element_type=jnp.float32)
    out_ref[...] = lg
    cm_ref[...] = jnp.max(lg.reshape(bt, NCH, CHUNK), axis=2)


def _tc_logf(x, centers, sigmas, ant_t, nb):
    bt = 128
    body = functools.partial(_tc_body, bt=bt, r=R)
    return pl.pallas_call(
        body,
        grid=(nb // bt,),
        in_specs=[
            pl.BlockSpec((bt, D), lambda i: (i, 0)),
            pl.BlockSpec((D, M), lambda i: (0, 0)),
            pl.BlockSpec((D, M), lambda i: (0, 0)),
            pl.BlockSpec((D, R), lambda i: (0, 0)),
        ],
        out_specs=[
            pl.BlockSpec((bt, R), lambda i: (i, 0)),
            pl.BlockSpec((bt, NCH), lambda i: (i, 0)),
        ],
        out_shape=[
            jax.ShapeDtypeStruct((nb, R), jnp.float32),
            jax.ShapeDtypeStruct((nb, NCH), jnp.float32),
        ],
        scratch_shapes=[pltpu.VMEM((D * MM, R), jnp.float32)],
        compiler_params=pltpu.CompilerParams(
            dimension_semantics=("arbitrary",)),
    )(x, centers, sigmas, ant_t)


CHUNK = 128
NCH = R // CHUNK


def _sc_body(lf_hbm, cm_hbm, oc_hbm, os_hbm, pc_hbm, out_hbm,
             buf0, buf1, cm0, cm1, pcv, ocv, osv, stabv, outv,
             sem0, sem1, semc0, semc1, *, rows_per_w):
    wid = lax.axis_index("s") * NC + lax.axis_index("c")
    base_row = wid * rows_per_w

    pltpu.sync_copy(oc_hbm, ocv)
    pltpu.sync_copy(os_hbm, osv)
    pltpu.sync_copy(pc_hbm, pcv)

    lane = lax.broadcasted_iota(jnp.int32, (L,), 0)
    seven = jnp.full((L,), 7, jnp.int32)

    s0vec = jnp.zeros((L,), jnp.float32)
    s1vec = jnp.zeros((L,), jnp.float32)
    ocvec = ocv[0:L]
    osvec = osv[0:L]
    for m in range(M_OUT):
        cm = _vtake(ocvec, jnp.full((L,), m, jnp.int32))
        sm = _vtake(osvec, jnp.full((L,), m, jnp.int32))
        inv = 0.5 / (sm * sm)
        e_acc = jnp.zeros((L,), jnp.float32)
        ue_acc = jnp.zeros((L,), jnp.float32)
        for g in range(7):
            p = lane + (16 * g)
            u = p.astype(jnp.float32) * (1.0 / (NPTS - 1))
            du = u - cm
            e = jnp.exp(-(du * du) * inv)
            e = jnp.where(p < NPTS, e, 0.0)
            e_acc = e_acc + e
            ue_acc = ue_acc + u * e
        s0vec = jnp.where(lane == m, _butterfly(e_acc, jnp.add), s0vec)
        s1vec = jnp.where(lane == m, _butterfly(ue_acc, jnp.add), s1vec)
    stabv[0:L] = s0vec
    stabv[L:2 * L] = s1vec

    def dma_row(local_row, buf, sem):
        return pltpu.make_async_copy(
            lf_hbm.at[base_row + local_row], buf, sem)

    def dma_cm(local_row, buf, sem):
        return pltpu.make_async_copy(
            cm_hbm.at[base_row + local_row], buf, sem)

    def merge_group(args):
        topk, topv, v, off = args
        pcg = pcv[pl.ds(off, L)]
        sk, sv = plsc.sort_key_val(v, pcg, descending=False)
        w = topk >= sk
        mk = jnp.where(w, topk, sk)
        mv = jnp.where(w, topv, sv)
        nk, nv = plsc.sort_key_val(mk, mv, descending=True)
        return nk, nv

    def scan_row(buf, cmaxv, local_row):
        work = [cmaxv[g * L:(g + 1) * L] for g in range(NCH // L)]
        theta = None
        for _ in range(TOP_N):
            mx = work[0]
            for wv in work[1:]:
                mx = jnp.maximum(mx, wv)
            mx = _butterfly(mx, jnp.maximum)
            work = [jnp.where(wv == mx, NEG, wv) for wv in work]
            theta = mx

        def scan_chunk(args):
            topk2, topv2, c = args
            for j in range(CHUNK // L):
                off = c * CHUNK + j * L
                v = buf[pl.ds(off, L)]
                hit = plsc.all_reduce_population_count(
                    v >= theta)[0] > 0
                topk2, topv2 = lax.cond(
                    hit, merge_group,
                    lambda a: (a[0], a[1]),
                    (topk2, topv2, v, off))
            return topk2, topv2

        def chunk_scan_body(c, carry):
            topk, topv = carry
            cm = plsc.load_gather(cmaxv, [jnp.full((L,), c, jnp.int32)])
            pred = cm[0] >= theta[0]
            topk, topv = lax.cond(
                pred, scan_chunk, lambda a: (a[0], a[1]),
                (topk, topv, c))
            return topk, topv

        topk0 = jnp.full((L,), NEG, jnp.float32)
        topv0 = jnp.full((L,), IBIG, jnp.int32)
        topk, topv = lax.fori_loop(
            0, NCH, chunk_scan_body, (topk0, topv0))

        selv = jnp.full((L,), NEG, jnp.float32)
        selp = jnp.full((L,), 0, jnp.int32)
        for n in range(TOP_N):
            mx = _butterfly(topk, jnp.maximum)
            cand = jnp.where(topk == mx, topv, IBIG)
            sel = _butterfly(cand, jnp.minimum)
            selv = jnp.where(lane == n, mx, selv)
            selp = jnp.where(lane == n, sel, selp)
            topk = jnp.where(topv == sel, NEG, topk)

        fv = jnp.exp(selv)
        cidx = selp & 7
        s0sel = plsc.load_gather(stabv, [cidx])
        s1sel = plsc.load_gather(stabv, [cidx + L])
        num = _butterfly(fv * s1sel, jnp.add)
        den = _butterfly(fv * s0sel, jnp.add) + EPS
        crisp = num / den
        plsc.store_scatter(outv, [jnp.full((L,), local_row, jnp.int32)],
                           crisp, mask=lane == 0)

    dma_row(0, buf0, sem0).start()
    dma_cm(0, cm0, semc0).start()
    dma_row(1, buf1, sem1).start()
    dma_cm(1, cm1, semc1).start()

    def pair_body(jj, carry):
        dma_row(2 * jj, buf0, sem0).wait()
        dma_cm(2 * jj, cm0, semc0).wait()
        scan_row(buf0, cm0, 2 * jj)

        @pl.when(jj < rows_per_w // 2 - 1)
        def _():
            dma_row(2 * jj + 2, buf0, sem0).start()
            dma_cm(2 * jj + 2, cm0, semc0).start()

        dma_row(2 * jj + 1, buf1, sem1).wait()
        dma_cm(2 * jj + 1, cm1, semc1).wait()
        scan_row(buf1, cm1, 2 * jj + 1)

        @pl.when(jj < rows_per_w // 2 - 1)
        def _():
            dma_row(2 * jj + 3, buf1, sem1).start()
            dma_cm(2 * jj + 3, cm1, semc1).start()

        return carry

    lax.fori_loop(0, rows_per_w // 2, pair_body, 0)

    pltpu.sync_copy(outv, out_hbm.at[pl.ds(base_row, rows_per_w)])


def _sc_topk(logf, cmax, oc, osg, pc, nb):
    rows_per_w = nb // NW
    mesh = plsc.VectorSubcoreMesh(core_axis_name="c", subcore_axis_name="s")
    f = pl.kernel(
        functools.partial(_sc_body, rows_per_w=rows_per_w),
        mesh=mesh,
        out_type=jax.ShapeDtypeStruct((nb,), jnp.float32),
        scratch_types=[
            pltpu.VMEM((R,), jnp.float32),
            pltpu.VMEM((R,), jnp.float32),
            pltpu.VMEM((NCH,), jnp.float32),
            pltpu.VMEM((NCH,), jnp.float32),
            pltpu.VMEM((R,), jnp.int32),
            pltpu.VMEM((16,), jnp.float32),
            pltpu.VMEM((16,), jnp.float32),
            pltpu.VMEM((32,), jnp.float32),
            pltpu.VMEM((rows_per_w,), jnp.float32),
            pltpu.SemaphoreType.DMA,
            pltpu.SemaphoreType.DMA,
            pltpu.SemaphoreType.DMA,
            pltpu.SemaphoreType.DMA,
        ],
        compiler_params=pltpu.CompilerParams(needs_layout_passes=False),
    )
    return f(logf, cmax, oc, osg, pc)


NSLICE = 4


def kernel(x, centers, sigmas, out_centers, out_sigmas, antecedents,
           consequents):
    ant_t = antecedents.T
    ridx = jnp.arange(R, dtype=jnp.int32)
    pc = ridx * 8 + consequents.astype(jnp.int32)
    oc = jnp.pad(out_centers.astype(jnp.float32), (0, 16 - M_OUT))
    osg = jnp.pad(out_sigmas.astype(jnp.float32), (0, 16 - M_OUT),
                  constant_values=1.0)

    nb = B // NSLICE
    outs = []
    for s in range(NSLICE):
        logf, cmax = _tc_logf(x[s * nb:(s + 1) * nb], centers, sigmas,
                              ant_t, nb)
        outs.append(_sc_topk(logf, cmax, oc, osg, pc, nb))
    return jnp.concatenate(outs)


# speedup vs baseline: 1.0840x; 1.0840x over previous
import functools

import jax
import jax.numpy as jnp
from jax import lax
from jax.experimental import pallas as pl
from jax.experimental.pallas import tpu as pltpu
from jax.experimental.pallas import tpu_sc as plsc

EPS = 1e-5
LOG_EPS = -11.512925464970229
NPTS = 100
TOP_N = 8
NEG = -1.0e30
IBIG = 2**30

NC, NS, L = 2, 16, 16
NW = NC * NS
B, D, M, R, M_OUT = 1024, 8, 5, 16384, 5
ROWS_PER_W = B // NW
MM = M + 1
NGRP = R // L
KSUP = 8
NSUP = NGRP // KSUP

_GATHER_DNUMS = lax.GatherDimensionNumbers(
    offset_dims=(), collapsed_slice_dims=(0,), start_index_map=(0,))


def _vtake(v, idx):
    return lax.gather(v, idx[:, None], _GATHER_DNUMS, (1,),
                      mode=lax.GatherScatterMode.PROMISE_IN_BOUNDS)


def _butterfly(v, op):
    lane = lax.broadcasted_iota(jnp.int32, (L,), 0)
    for sh in (8, 4, 2, 1):
        v = op(v, _vtake(v, lane ^ sh))
    return v


def _tc_body(x_ref, cs_ref, ss_ref, ant_ref, out_ref, cm_ref, oh_ref,
             *, bt, r):
    @pl.when(pl.program_id(0) == 0)
    def _build_onehot():
        for i in range(D):
            ai = ant_ref[i:i + 1, :]
            ai = jnp.where(ai < 0, M, ai)
            js = jax.lax.broadcasted_iota(jnp.int32, (MM, r), 0)
            oh_ref[i * MM:(i + 1) * MM, :] = (js == ai).astype(jnp.float32)

    x = x_ref[...]
    pieces = []
    for i in range(D):
        xi = x[:, i:i + 1]
        ci = cs_ref[i:i + 1, :]
        si = ss_ref[i:i + 1, :]
        t = -((xi - ci) ** 2) / (2.0 * si * si)
        t = jnp.clip(t, LOG_EPS, 0.0)
        pieces.append(t)
        pieces.append(jnp.zeros((bt, 1), jnp.float32))
    logF = jnp.concatenate(pieces, axis=1)

    lg = jax.lax.dot_general(
        logF, oh_ref[...], (((1,), (0,)), ((), ())),
        precision=jax.lax.Precision.HIGHEST,
        preferred_element_type=jnp.float32)
    out_ref[...] = lg
    cm_ref[...] = jnp.max(lg.reshape(bt, NCH, CHUNK), axis=2)


def _tc_logf(x, centers, sigmas, ant_t, nb):
    bt = 128
    body = functools.partial(_tc_body, bt=bt, r=R)
    return pl.pallas_call(
        body,
        grid=(nb // bt,),
        in_specs=[
            pl.BlockSpec((bt, D), lambda i: (i, 0)),
            pl.BlockSpec((D, M), lambda i: (0, 0)),
            pl.BlockSpec((D, M), lambda i: (0, 0)),
            pl.BlockSpec((D, R), lambda i: (0, 0)),
        ],
        out_specs=[
            pl.BlockSpec((bt, R), lambda i: (i, 0)),
            pl.BlockSpec((bt, NCH), lambda i: (i, 0)),
        ],
        out_shape=[
            jax.ShapeDtypeStruct((nb, R), jnp.float32),
            jax.ShapeDtypeStruct((nb, NCH), jnp.float32),
        ],
        scratch_shapes=[pltpu.VMEM((D * MM, R), jnp.float32)],
        compiler_params=pltpu.CompilerParams(
            dimension_semantics=("arbitrary",)),
    )(x, centers, sigmas, ant_t)


CHUNK = 128
NCH = R // CHUNK


def _sc_body(lf_hbm, cm_hbm, oc_hbm, os_hbm, pc_hbm, out_hbm,
             buf0, buf1, cm0, cm1, pcv, ocv, osv, stabv, outv,
             sem0, sem1, semc0, semc1, *, rows_per_w):
    wid = lax.axis_index("s") * NC + lax.axis_index("c")
    base_row = wid * rows_per_w

    pltpu.sync_copy(oc_hbm, ocv)
    pltpu.sync_copy(os_hbm, osv)
    pltpu.sync_copy(pc_hbm, pcv)

    lane = lax.broadcasted_iota(jnp.int32, (L,), 0)
    seven = jnp.full((L,), 7, jnp.int32)

    s0vec = jnp.zeros((L,), jnp.float32)
    s1vec = jnp.zeros((L,), jnp.float32)
    ocvec = ocv[0:L]
    osvec = osv[0:L]
    for m in range(M_OUT):
        cm = _vtake(ocvec, jnp.full((L,), m, jnp.int32))
        sm = _vtake(osvec, jnp.full((L,), m, jnp.int32))
        inv = 0.5 / (sm * sm)
        e_acc = jnp.zeros((L,), jnp.float32)
        ue_acc = jnp.zeros((L,), jnp.float32)
        for g in range(7):
            p = lane + (16 * g)
            u = p.astype(jnp.float32) * (1.0 / (NPTS - 1))
            du = u - cm
            e = jnp.exp(-(du * du) * inv)
            e = jnp.where(p < NPTS, e, 0.0)
            e_acc = e_acc + e
            ue_acc = ue_acc + u * e
        s0vec = jnp.where(lane == m, _butterfly(e_acc, jnp.add), s0vec)
        s1vec = jnp.where(lane == m, _butterfly(ue_acc, jnp.add), s1vec)
    stabv[0:L] = s0vec
    stabv[L:2 * L] = s1vec

    def dma_row(local_row, buf, sem):
        return pltpu.make_async_copy(
            lf_hbm.at[base_row + local_row], buf, sem)

    def dma_cm(local_row, buf, sem):
        return pltpu.make_async_copy(
            cm_hbm.at[base_row + local_row], buf, sem)

    def merge_group(args):
        topk, topv, v, off = args
        pcg = pcv[pl.ds(off, L)]
        sk, sv = plsc.sort_key_val(v, pcg, descending=False)
        w = topk >= sk
        mk = jnp.where(w, topk, sk)
        mv = jnp.where(w, topv, sv)
        nk, nv = plsc.sort_key_val(mk, mv, descending=True)
        return nk, nv

    def scan_row(buf, cmaxv, local_row):
        work = [cmaxv[g * L:(g + 1) * L] for g in range(NCH // L)]
        theta = None
        for _ in range(TOP_N):
            mx = work[0]
            for wv in work[1:]:
                mx = jnp.maximum(mx, wv)
            mx = _butterfly(mx, jnp.maximum)
            work = [jnp.where(wv == mx, NEG, wv) for wv in work]
            theta = mx

        def scan_chunk(args):
            topk2, topv2, c = args
            for j in range(CHUNK // L):
                off = c * CHUNK + j * L
                v = buf[pl.ds(off, L)]
                hit = plsc.all_reduce_population_count(
                    v >= theta)[0] > 0
                topk2, topv2 = lax.cond(
                    hit, merge_group,
                    lambda a: (a[0], a[1]),
                    (topk2, topv2, v, off))
            return topk2, topv2

        def chunk_scan_body(c, carry):
            topk, topv = carry
            cm = plsc.load_gather(cmaxv, [jnp.full((L,), c, jnp.int32)])
            pred = cm[0] >= theta[0]
            topk, topv = lax.cond(
                pred, scan_chunk, lambda a: (a[0], a[1]),
                (topk, topv, c))
            return topk, topv

        topk0 = jnp.full((L,), NEG, jnp.float32)
        topv0 = jnp.full((L,), IBIG, jnp.int32)
        topk, topv = lax.fori_loop(
            0, NCH, chunk_scan_body, (topk0, topv0))

        selv = jnp.full((L,), NEG, jnp.float32)
        selp = jnp.full((L,), 0, jnp.int32)
        for n in range(TOP_N):
            mx = _butterfly(topk, jnp.maximum)
            cand = jnp.where(topk == mx, topv, IBIG)
            sel = _butterfly(cand, jnp.minimum)
            selv = jnp.where(lane == n, mx, selv)
            selp = jnp.where(lane == n, sel, selp)
            topk = jnp.where(topv == sel, NEG, topk)

        fv = jnp.exp(selv)
        cidx = selp & 7
        s0sel = plsc.load_gather(stabv, [cidx])
        s1sel = plsc.load_gather(stabv, [cidx + L])
        num = _butterfly(fv * s1sel, jnp.add)
        den = _butterfly(fv * s0sel, jnp.add) + EPS
        crisp = num / den
        plsc.store_scatter(outv, [jnp.full((L,), local_row, jnp.int32)],
                           crisp, mask=lane == 0)

    dma_row(0, buf0, sem0).start()
    dma_cm(0, cm0, semc0).start()
    dma_row(1, buf1, sem1).start()
    dma_cm(1, cm1, semc1).start()

    def pair_body(jj, carry):
        dma_row(2 * jj, buf0, sem0).wait()
        dma_cm(2 * jj, cm0, semc0).wait()
        scan_row(buf0, cm0, 2 * jj)

        @pl.when(jj < rows_per_w // 2 - 1)
        def _():
            dma_row(2 * jj + 2, buf0, sem0).start()
            dma_cm(2 * jj + 2, cm0, semc0).start()

        dma_row(2 * jj + 1, buf1, sem1).wait()
        dma_cm(2 * jj + 1, cm1, semc1).wait()
        scan_row(buf1, cm1, 2 * jj + 1)

        @pl.when(jj < rows_per_w // 2 - 1)
        def _():
            dma_row(2 * jj + 3, buf1, sem1).start()
            dma_cm(2 * jj + 3, cm1, semc1).start()

        return carry

    lax.fori_loop(0, rows_per_w // 2, pair_body, 0)

    pltpu.sync_copy(outv, out_hbm.at[pl.ds(base_row, rows_per_w)])


def _sc_topk(logf, cmax, oc, osg, pc, nb):
    rows_per_w = nb // NW
    mesh = plsc.VectorSubcoreMesh(core_axis_name="c", subcore_axis_name="s")
    f = pl.kernel(
        functools.partial(_sc_body, rows_per_w=rows_per_w),
        mesh=mesh,
        out_type=jax.ShapeDtypeStruct((nb,), jnp.float32),
        scratch_types=[
            pltpu.VMEM((R,), jnp.float32),
            pltpu.VMEM((R,), jnp.float32),
            pltpu.VMEM((NCH,), jnp.float32),
            pltpu.VMEM((NCH,), jnp.float32),
            pltpu.VMEM((R,), jnp.int32),
            pltpu.VMEM((16,), jnp.float32),
            pltpu.VMEM((16,), jnp.float32),
            pltpu.VMEM((32,), jnp.float32),
            pltpu.VMEM((rows_per_w,), jnp.float32),
            pltpu.SemaphoreType.DMA,
            pltpu.SemaphoreType.DMA,
            pltpu.SemaphoreType.DMA,
            pltpu.SemaphoreType.DMA,
        ],
        compiler_params=pltpu.CompilerParams(needs_layout_passes=False),
    )
    return f(logf, cmax, oc, osg, pc)


NSLICE = 2


def kernel(x, centers, sigmas, out_centers, out_sigmas, antecedents,
           consequents):
    ant_t = antecedents.T
    ridx = jnp.arange(R, dtype=jnp.int32)
    pc = ridx * 8 + consequents.astype(jnp.int32)
    oc = jnp.pad(out_centers.astype(jnp.float32), (0, 16 - M_OUT))
    osg = jnp.pad(out_sigmas.astype(jnp.float32), (0, 16 - M_OUT),
                  constant_values=1.0)

    nb = B // NSLICE
    outs = []
    for s in range(NSLICE):
        logf, cmax = _tc_logf(x[s * nb:(s + 1) * nb], centers, sigmas,
                              ant_t, nb)
        outs.append(_sc_topk(logf, cmax, oc, osg, pc, nb))
    return jnp.concatenate(outs)
